# Initial kernel scaffold; baseline (speedup 1.0000x reference)
import jax
import jax.numpy as jnp
from jax.experimental import pallas as pl


def kernel(x, adj_t, Wl0, bl0, Wr0, Wl1, bl1, Wr1, Wl2, bl2, Wr2, g0, be0, g1, be1, g2, be2, Wf, bf):
    raise NotImplementedError("write your pallas kernel here")


# speedup vs baseline: 3.7213x; 3.7213x over previous
import math

import jax
import jax.numpy as jnp
from jax import lax
from jax.experimental import pallas as pl
from jax.experimental.pallas import tpu as pltpu
from jax.experimental.pallas import tpu_sc as plsc

_BN_SCALE = 1.0 / math.sqrt(1.0 + 1e-5)
_K = 128


def _sc_geometry(E):
  info = plsc.get_sparse_core_info()
  NC, NS = info.num_cores, info.num_subcores
  NW = NC * NS
  iters = -(-E // (_K * NW))
  return NC, NS, NW, iters, iters * _K * NW


def _make_sc_agg(N, D, E_pad, mode):
  NC, NS, NW, iters, e_chk = _sc_geometry(E_pad)
  assert e_chk == E_pad
  NA = N + 8
  rows_per_tile = (N // NS) & ~7
  rows_extra = N - rows_per_tile * NS
  assert rows_extra % 8 == 0 and rows_extra + 8 <= _K
  gather = mode == "agg"

  mesh = plsc.VectorSubcoreMesh(core_axis_name="c", subcore_axis_name="s")
  out_type = [jax.ShapeDtypeStruct((NC, N, D), jnp.float32)]
  scratch = [
      pltpu.VMEM((_K,), jnp.int32),
      pltpu.VMEM((_K, D), jnp.float32),
      pltpu.VMEM_SHARED((NA, D), jnp.float32),
  ]
  if gather:
    scratch.append(pltpu.VMEM((_K,), jnp.int32))
    scratch.append(pltpu.SemaphoreType.DMA)

  def body(*refs):
    if gather:
      h_hbm, src_hbm, dst_hbm, out_hbm, dst_v, rows_v, acc, src_v, gsem = refs
    else:
      dst_hbm, out_hbm, dst_v, rows_v, acc = refs
    cid = lax.axis_index("c")
    sid = lax.axis_index("s")
    wid = sid * NC + cid

    def fill_rows(val):
      vec = jnp.full((16,), val, jnp.float32)

      def w(i, _):
        rows_v[i // (D // 16), pl.ds((i % (D // 16)) * 16, 16)] = vec
        return 0
      lax.fori_loop(0, _K * (D // 16), w, 0)

    fill_rows(0.0)

    row0 = sid * rows_per_tile
    n_full = rows_per_tile // _K
    tail = rows_per_tile - n_full * _K
    for b in range(n_full):
      pltpu.sync_copy(rows_v, acc.at[pl.ds(row0 + b * _K, _K)])
    if tail:
      pltpu.sync_copy(rows_v.at[pl.ds(0, tail)],
                      acc.at[pl.ds(row0 + n_full * _K, tail)])

    @pl.when(sid == 0)
    def _():
      pltpu.sync_copy(rows_v.at[pl.ds(0, rows_extra + 8)],
                      acc.at[pl.ds(rows_per_tile * NS, rows_extra + 8)])

    if not gather:
      fill_rows(1.0)
    plsc.subcore_barrier()

    def chunk(i, _):
      base = (i * NW + wid) * _K
      pltpu.sync_copy(dst_hbm.at[pl.ds(base, _K)], dst_v)
      if gather:
        pltpu.sync_copy(src_hbm.at[pl.ds(base, _K)], src_v)
        pltpu.async_copy(h_hbm.at[src_v], rows_v, gsem).wait()
      pltpu.sync_copy(rows_v, acc.at[dst_v], add=True)
      return 0
    lax.fori_loop(0, iters, chunk, 0)
    plsc.subcore_barrier()

    pltpu.sync_copy(acc.at[pl.ds(row0, rows_per_tile)],
                    out_hbm.at[cid, pl.ds(row0, rows_per_tile)])
    if rows_extra:
      @pl.when(sid == 0)
      def _():
        pltpu.sync_copy(acc.at[pl.ds(rows_per_tile * NS, rows_extra)],
                        out_hbm.at[cid, pl.ds(rows_per_tile * NS, rows_extra)])

  return pl.kernel(body, out_type=out_type, mesh=mesh, scratch_types=scratch)


def _dense_layer(p, cnt, h, Wl, bl, Wr, g, be, relu):
  N, D = h.shape
  TILE = 1000

  def body(p0_ref, p1_ref, c0_ref, c1_ref, h_ref, wl_ref, bl_ref, wr_ref,
           g_ref, be_ref, o_ref):
    cnt_t = c0_ref[:, :1] + c1_ref[:, :1]
    inv = 1.0 / jnp.maximum(cnt_t, 1.0)
    mean = (p0_ref[...] + p1_ref[...]) * inv
    acc = lax.dot_general(mean, wl_ref[...], (((1,), (1,)), ((), ())),
                          preferred_element_type=jnp.float32)
    acc = acc + lax.dot_general(h_ref[...], wr_ref[...],
                                (((1,), (1,)), ((), ())),
                                preferred_element_type=jnp.float32)
    acc = acc + bl_ref[...]
    if relu:
      acc = jnp.maximum(acc, 0.0)
    o_ref[...] = acc * (g_ref[...] * _BN_SCALE) + be_ref[...]

  grid = (N // TILE,)
  row_spec = pl.BlockSpec((TILE, D), lambda i: (i, 0))
  cnt_spec = pl.BlockSpec((TILE, D), lambda i: (i, 0))
  full = lambda shape: pl.BlockSpec(shape, lambda i: (0,) * len(shape))
  return pl.pallas_call(
      body,
      grid=grid,
      in_specs=[row_spec, row_spec, cnt_spec, cnt_spec, row_spec,
                full((D, D)), full((1, D)), full((D, D)),
                full((1, D)), full((1, D))],
      out_specs=row_spec,
      out_shape=jax.ShapeDtypeStruct((N, D), jnp.float32),
  )(p[0], p[1], cnt[0], cnt[1], h, Wl, bl.reshape(1, D), Wr,
    g.reshape(1, D), be.reshape(1, D))


def _dense_final(p, cnt, h, Wl, bl, Wr, g, be, Wf, bf):
  N, D = h.shape
  TILE = 1000

  def body(p0_ref, p1_ref, c0_ref, c1_ref, h_ref, wl_ref, bl_ref, wr_ref,
           g_ref, be_ref, wf_ref, bf_ref, o_ref):
    cnt_t = c0_ref[:, :1] + c1_ref[:, :1]
    inv = 1.0 / jnp.maximum(cnt_t, 1.0)
    mean = (p0_ref[...] + p1_ref[...]) * inv
    acc = lax.dot_general(mean, wl_ref[...], (((1,), (1,)), ((), ())),
                          preferred_element_type=jnp.float32)
    acc = acc + lax.dot_general(h_ref[...], wr_ref[...],
                                (((1,), (1,)), ((), ())),
                                preferred_element_type=jnp.float32)
    acc = acc + bl_ref[...]
    acc = acc * (g_ref[...] * _BN_SCALE) + be_ref[...]
    logit = jnp.sum(acc * wf_ref[...], axis=1, keepdims=True) + bf_ref[0, 0]
    o_ref[...] = 1.0 / (1.0 + jnp.exp(-logit))

  grid = (N // TILE,)
  row_spec = pl.BlockSpec((TILE, D), lambda i: (i, 0))
  cnt_spec = pl.BlockSpec((TILE, D), lambda i: (i, 0))
  full = lambda shape: pl.BlockSpec(shape, lambda i: (0,) * len(shape))
  return pl.pallas_call(
      body,
      grid=grid,
      in_specs=[row_spec, row_spec, cnt_spec, cnt_spec, row_spec,
                full((D, D)), full((1, D)), full((D, D)),
                full((1, D)), full((1, D)), full((1, D)),
                pl.BlockSpec(memory_space=pltpu.SMEM)],
      out_specs=pl.BlockSpec((TILE, 1), lambda i: (i, 0)),
      out_shape=jax.ShapeDtypeStruct((N, 1), jnp.float32),
  )(p[0], p[1], cnt[0], cnt[1], h, Wl, bl.reshape(1, D), Wr,
    g.reshape(1, D), be.reshape(1, D), Wf.reshape(1, D), bf.reshape(1, 1))


def _pad_edges(src, dst, N, E_pad):
  pad = E_pad - src.shape[0]
  if pad == 0:
    return src, dst
  src_p = jnp.concatenate([src, jnp.zeros((pad,), jnp.int32)])
  dst_p = jnp.concatenate([dst, jnp.full((pad,), N, jnp.int32)])
  return src_p, dst_p


@jax.jit
def kernel(x, adj_t, Wl0, bl0, Wr0, Wl1, bl1, Wr1, Wl2, bl2, Wr2,
           g0, be0, g1, be1, g2, be2, Wf, bf):
  N, D = x.shape
  E = adj_t.shape[1]
  _, _, _, _, E_pad = _sc_geometry(E)
  src, dst = _pad_edges(adj_t[0], adj_t[1], N, E_pad)

  count = _make_sc_agg(N, D, E_pad, mode="count")
  agg = _make_sc_agg(N, D, E_pad, mode="agg")

  (cnt,) = count(dst)
  (p,) = agg(x, src, dst)
  h1 = _dense_layer(p, cnt, x, Wl0, bl0, Wr0, g0, be0, relu=True)
  (p,) = agg(h1, src, dst)
  h2 = _dense_layer(p, cnt, h1, Wl1, bl1, Wr1, g1, be1, relu=True)
  (p,) = agg(h2, src, dst)
  return _dense_final(p, cnt, h2, Wl2, bl2, Wr2, g2, be2, Wf, bf)
